# baseline (speedup 1.0000x reference)
import jax
import jax.numpy as jnp
from jax import lax
from jax.experimental import pallas as pl
from jax.experimental.pallas import tpu as pltpu

_L = 256
_H = _W = 256
_NB = 180
_R = 32
_NCHUNK = _H // _R
_NSTEP = 9
_BPS = 20

_F8 = jnp.float8_e4m3fn


def _quantize(band):
    lo = jnp.min(band)
    hi = jnp.max(band)
    scaled = (band - lo) / jnp.maximum(hi - lo, jnp.float32(1e-12))
    return jnp.clip(jnp.round(scaled * 255.0), 0.0, 255.0)


def _onehot_u8(x, iota_lvl):
    m = x[:, :, None] == iota_lvl
    return jnp.where(m, _F8(1.0), _F8(0.0)).reshape(_R * _W, _L)


def _onehot_t(v, iota_sub):
    return jnp.where(iota_sub == v, jnp.bfloat16(1.0), jnp.bfloat16(0.0))


def _rollm1(v):
    return jnp.concatenate([v[:, 1:], v[:, :1]], axis=1)


def _rollp1(v):
    return jnp.concatenate([v[:, -1:], v[:, :-1]], axis=1)


def _small_dot(a_t, b_t):
    return lax.dot_general(
        a_t, b_t, (((1,), (1,)), ((), ())),
        preferred_element_type=jnp.float32,
    )


def _in_copy(img_ref, buf_ref, in_sem, blk):
    return pltpu.make_async_copy(
        img_ref.at[pl.ds(blk * _BPS, _BPS)], buf_ref.at[blk % 2],
        in_sem.at[blk % 2])


def _out_copy(out_ref, buf_ref, out_sem, blk):
    return pltpu.make_async_copy(
        buf_ref.at[blk % 2], out_ref.at[pl.ds(blk * _BPS, _BPS)],
        out_sem.at[blk % 2])


def _fused_kernel(idx_ref, band_ref, img_ref, out_ref,
                  sc_ref, sct_ref, acc0_ref, acc1_ref, acc2_ref, acc3_ref,
                  buf_ref, var_ref, in_sem, out_sem, var_sem):
    acc_refs = (acc0_ref, acc1_ref, acc2_ref, acc3_ref)
    del idx_ref
    j = pl.program_id(0)

    @pl.when(j == 0)
    def _start_first():
        _in_copy(img_ref, buf_ref, in_sem, 0).start()

    @pl.when(j >= 1)
    def _wait_prev_out():
        _out_copy(out_ref, buf_ref, out_sem, j - 1).wait()

    @pl.when(j <= _NSTEP - 2)
    def _start_next_in():
        _in_copy(img_ref, buf_ref, in_sem, j + 1).start()

    _in_copy(img_ref, buf_ref, in_sem, j).wait()
    _out_copy(out_ref, buf_ref, out_sem, j).start()

    @pl.when(j == 0)
    def _init():
        codes = _quantize(band_ref[0])
        sc_ref[0] = codes.astype(jnp.uint8)
        down = jnp.concatenate([codes[1:, :], codes[:1, :]], axis=0)
        sc_ref[1] = down.astype(jnp.uint8)
        sct_ref[...] = codes.T.astype(jnp.bfloat16)
        for r in acc_refs:
            r[...] = jnp.zeros_like(r)

    @pl.when(j < _NCHUNK)
    def _chunk():
        iota_lvl = lax.broadcasted_iota(jnp.uint8, (_R, _W, _L), 2)
        off = pl.multiple_of(j * _R, _R)
        a = sc_ref[0, pl.ds(off, _R), :]
        d = sc_ref[1, pl.ds(off, _R), :]
        b01 = jnp.concatenate([a[:, 1:], a[:, :1]], axis=1)
        b11 = jnp.concatenate([d[:, 1:], d[:, :1]], axis=1)
        b1m = jnp.concatenate([d[:, -1:], d[:, :-1]], axis=1)
        oh_a = _onehot_u8(a, iota_lvl)
        for r, b in zip(acc_refs, (b01, b11, d, b1m)):
            r[...] += lax.dot_general(
                oh_a, _onehot_u8(b, iota_lvl), (((0,), (0,)), ((), ())),
                preferred_element_type=jnp.float32,
            )

    @pl.when(j == _NSTEP - 1)
    def _finish():
        u8b = sc_ref[0]
        row0 = u8b[0:1, :].astype(jnp.bfloat16)
        row255 = u8b[255:256, :].astype(jnp.bfloat16)
        col0 = sct_ref[0:1, :]
        col255 = sct_ref[255:256, :]
        iota_sub = lax.broadcasted_iota(jnp.int32, (_L, _W), 0).astype(jnp.bfloat16)
        oh_row0_m1 = _onehot_t(_rollm1(row0), iota_sub)
        oh_row0_p1 = _onehot_t(_rollp1(row0), iota_sub)
        oh_row255 = _onehot_t(row255, iota_sub)
        lane = lax.broadcasted_iota(jnp.int32, (_L, _W), 1)
        oh_r255_no_last = jnp.where(lane < 255, oh_row255, jnp.bfloat16(0.0))
        oh_r255_no_first = jnp.where(lane > 0, oh_row255, jnp.bfloat16(0.0))
        corr0 = _small_dot(_onehot_t(col255, iota_sub), _onehot_t(col0, iota_sub))
        corr1 = _small_dot(_onehot_t(col255, iota_sub),
                           _onehot_t(_rollm1(col0), iota_sub))
        corr1 += _small_dot(oh_r255_no_last, oh_row0_m1)
        corr2 = _small_dot(oh_row255, _onehot_t(row0, iota_sub))
        corr3 = _small_dot(_onehot_t(col0, iota_sub),
                           _onehot_t(_rollm1(col255), iota_sub))
        corr3 += _small_dot(oh_r255_no_first, oh_row0_p1)

        c0 = acc0_ref[...] - corr0
        c1 = acc1_ref[...] - corr1
        c2 = acc2_ref[...] - corr2
        c3 = acc3_ref[...] - corr3
        mean = (c0 + c1 + c2 + c3) * 0.25
        d0 = c0 - mean
        d1 = c1 - mean
        d2 = c2 - mean
        d3 = c3 - mean
        var_ref[0] = (d0 * d0 + d1 * d1 + d2 * d2 + d3 * d3) * 0.25
        vcp = pltpu.make_async_copy(
            var_ref, out_ref.at[pl.ds(_NB, 1)], var_sem)
        vcp.start()
        vcp.wait()
        _out_copy(out_ref, buf_ref, out_sem, _NSTEP - 1).wait()


def kernel(image, index):
    idx = jnp.asarray(index, jnp.int32).reshape(1)
    return pl.pallas_call(
        _fused_kernel,
        out_shape=jax.ShapeDtypeStruct((_NB + 1, _H, _W), jnp.float32),
        grid_spec=pltpu.PrefetchScalarGridSpec(
            num_scalar_prefetch=1,
            grid=(_NSTEP,),
            in_specs=[
                pl.BlockSpec((1, _H, _W), lambda j, i: (i[0], 0, 0)),
                pl.BlockSpec(memory_space=pl.ANY),
            ],
            out_specs=pl.BlockSpec(memory_space=pl.ANY),
            scratch_shapes=[
                pltpu.VMEM((2, _H, _W), jnp.uint8),
                pltpu.VMEM((_H, _W), jnp.bfloat16),
                pltpu.VMEM((_L, _L), jnp.float32),
                pltpu.VMEM((_L, _L), jnp.float32),
                pltpu.VMEM((_L, _L), jnp.float32),
                pltpu.VMEM((_L, _L), jnp.float32),
                pltpu.VMEM((2, _BPS, _H, _W), jnp.float32),
                pltpu.VMEM((1, _H, _W), jnp.float32),
                pltpu.SemaphoreType.DMA((2,)),
                pltpu.SemaphoreType.DMA((2,)),
                pltpu.SemaphoreType.DMA,
            ],
        ),
        compiler_params=pltpu.CompilerParams(
            dimension_semantics=("arbitrary",),
            vmem_limit_bytes=56 * 1024 * 1024,
        ),
        name="glcm_append_fused",
    )(idx, image, image)


# speedup vs baseline: 1.0508x; 1.0508x over previous
import jax
import jax.numpy as jnp
from jax import lax
from jax.experimental import pallas as pl
from jax.experimental.pallas import tpu as pltpu

_L = 256
_H = _W = 256
_NB = 180
_R = 64
_NCHUNK = _H // _R
_NSTEP = 5
_BPS = 36

_F8 = jnp.float8_e4m3fn


def _quantize(band):
    lo = jnp.min(band)
    hi = jnp.max(band)
    scaled = (band - lo) / jnp.maximum(hi - lo, jnp.float32(1e-12))
    return jnp.clip(jnp.round(scaled * 255.0), 0.0, 255.0)


def _onehot_u8(x, iota_lvl):
    m = x[:, :, None] != iota_lvl
    return jnp.where(m, _F8(0.0), _F8(1.0)).reshape(_R * _W, _L)


def _onehot_t(v, iota_sub):
    return jnp.where(iota_sub == v, jnp.bfloat16(1.0), jnp.bfloat16(0.0))


def _rollm1(v):
    return jnp.concatenate([v[:, 1:], v[:, :1]], axis=1)


def _rollp1(v):
    return jnp.concatenate([v[:, -1:], v[:, :-1]], axis=1)


def _small_dot(a_t, b_t):
    return lax.dot_general(
        a_t, b_t, (((1,), (1,)), ((), ())),
        preferred_element_type=jnp.float32,
    )


def _in_copy(img_ref, buf_ref, in_sem, blk):
    return pltpu.make_async_copy(
        img_ref.at[pl.ds(blk * _BPS, _BPS)], buf_ref.at[blk % 2],
        in_sem.at[blk % 2])


def _out_copy(out_ref, buf_ref, out_sem, blk):
    return pltpu.make_async_copy(
        buf_ref.at[blk % 2], out_ref.at[pl.ds(blk * _BPS, _BPS)],
        out_sem.at[blk % 2])


def _fused_kernel(idx_ref, band_ref, img_ref, out_ref,
                  sc_ref, sct_ref, acc0_ref, acc1_ref, acc2_ref, acc3_ref,
                  buf_ref, var_ref, in_sem, out_sem, var_sem):
    acc_refs = (acc0_ref, acc1_ref, acc2_ref, acc3_ref)
    del idx_ref
    j = pl.program_id(0)

    @pl.when(j == 0)
    def _start_first():
        _in_copy(img_ref, buf_ref, in_sem, 0).start()

    @pl.when(j >= 1)
    def _wait_prev_out():
        _out_copy(out_ref, buf_ref, out_sem, j - 1).wait()

    @pl.when(j <= _NSTEP - 2)
    def _start_next_in():
        _in_copy(img_ref, buf_ref, in_sem, j + 1).start()

    _in_copy(img_ref, buf_ref, in_sem, j).wait()
    _out_copy(out_ref, buf_ref, out_sem, j).start()

    @pl.when(j == 0)
    def _init():
        codes = _quantize(band_ref[0])
        sc_ref[0] = codes.astype(jnp.uint8)
        down = jnp.concatenate([codes[1:, :], codes[:1, :]], axis=0)
        sc_ref[1] = down.astype(jnp.uint8)
        sct_ref[...] = codes.T.astype(jnp.bfloat16)
        for r in acc_refs:
            r[...] = jnp.zeros_like(r)

    @pl.when(j < _NCHUNK)
    def _chunk():
        iota_lvl = lax.broadcasted_iota(jnp.uint8, (_R, _W, _L), 2)
        off = pl.multiple_of(j * _R, _R)
        a = sc_ref[0, pl.ds(off, _R), :]
        d = sc_ref[1, pl.ds(off, _R), :]
        b01 = jnp.concatenate([a[:, 1:], a[:, :1]], axis=1)
        b11 = jnp.concatenate([d[:, 1:], d[:, :1]], axis=1)
        b1m = jnp.concatenate([d[:, -1:], d[:, :-1]], axis=1)
        oh_a = _onehot_u8(a, iota_lvl)
        for r, b in zip(acc_refs, (b01, b11, d, b1m)):
            r[...] += lax.dot_general(
                oh_a, _onehot_u8(b, iota_lvl), (((0,), (0,)), ((), ())),
                preferred_element_type=jnp.float32,
            )

    @pl.when(j == _NSTEP - 1)
    def _finish():
        u8b = sc_ref[0]
        row0 = u8b[0:1, :].astype(jnp.bfloat16)
        row255 = u8b[255:256, :].astype(jnp.bfloat16)
        col0 = sct_ref[0:1, :]
        col255 = sct_ref[255:256, :]
        iota_sub = lax.broadcasted_iota(jnp.int32, (_L, _W), 0).astype(jnp.bfloat16)
        oh_row0_m1 = _onehot_t(_rollm1(row0), iota_sub)
        oh_row0_p1 = _onehot_t(_rollp1(row0), iota_sub)
        oh_row255 = _onehot_t(row255, iota_sub)
        lane = lax.broadcasted_iota(jnp.int32, (_L, _W), 1)
        oh_r255_no_last = jnp.where(lane < 255, oh_row255, jnp.bfloat16(0.0))
        oh_r255_no_first = jnp.where(lane > 0, oh_row255, jnp.bfloat16(0.0))
        corr0 = _small_dot(_onehot_t(col255, iota_sub), _onehot_t(col0, iota_sub))
        corr1 = _small_dot(_onehot_t(col255, iota_sub),
                           _onehot_t(_rollm1(col0), iota_sub))
        corr1 += _small_dot(oh_r255_no_last, oh_row0_m1)
        corr2 = _small_dot(oh_row255, _onehot_t(row0, iota_sub))
        corr3 = _small_dot(_onehot_t(col0, iota_sub),
                           _onehot_t(_rollm1(col255), iota_sub))
        corr3 += _small_dot(oh_r255_no_first, oh_row0_p1)

        c0 = acc0_ref[...] - corr0
        c1 = acc1_ref[...] - corr1
        c2 = acc2_ref[...] - corr2
        c3 = acc3_ref[...] - corr3
        mean = (c0 + c1 + c2 + c3) * 0.25
        d0 = c0 - mean
        d1 = c1 - mean
        d2 = c2 - mean
        d3 = c3 - mean
        var_ref[0] = (d0 * d0 + d1 * d1 + d2 * d2 + d3 * d3) * 0.25
        vcp = pltpu.make_async_copy(
            var_ref, out_ref.at[pl.ds(_NB, 1)], var_sem)
        vcp.start()
        vcp.wait()
        _out_copy(out_ref, buf_ref, out_sem, _NSTEP - 1).wait()


def kernel(image, index):
    idx = jnp.asarray(index, jnp.int32).reshape(1)
    return pl.pallas_call(
        _fused_kernel,
        out_shape=jax.ShapeDtypeStruct((_NB + 1, _H, _W), jnp.float32),
        grid_spec=pltpu.PrefetchScalarGridSpec(
            num_scalar_prefetch=1,
            grid=(_NSTEP,),
            in_specs=[
                pl.BlockSpec((1, _H, _W), lambda j, i: (i[0], 0, 0)),
                pl.BlockSpec(memory_space=pl.ANY),
            ],
            out_specs=pl.BlockSpec(memory_space=pl.ANY),
            scratch_shapes=[
                pltpu.VMEM((2, _H, _W), jnp.uint8),
                pltpu.VMEM((_H, _W), jnp.bfloat16),
                pltpu.VMEM((_L, _L), jnp.float32),
                pltpu.VMEM((_L, _L), jnp.float32),
                pltpu.VMEM((_L, _L), jnp.float32),
                pltpu.VMEM((_L, _L), jnp.float32),
                pltpu.VMEM((2, _BPS, _H, _W), jnp.float32),
                pltpu.VMEM((1, _H, _W), jnp.float32),
                pltpu.SemaphoreType.DMA((2,)),
                pltpu.SemaphoreType.DMA((2,)),
                pltpu.SemaphoreType.DMA,
            ],
        ),
        compiler_params=pltpu.CompilerParams(
            dimension_semantics=("arbitrary",),
            vmem_limit_bytes=56 * 1024 * 1024,
        ),
        name="glcm_append_fused",
    )(idx, image, image)
